# Initial kernel scaffold; baseline (speedup 1.0000x reference)
import jax
import jax.numpy as jnp
from jax.experimental import pallas as pl


def kernel(x, edge_index, edge_weight, g_size, W_emb, b_emb, gc1_T0, gc1_T1, gc1_b, gc1_gamma, gc1_beta, gc2_T0, gc2_T1, gc2_b, gc2_gamma, gc2_beta, gcl_T0, gcl_T1):
    raise NotImplementedError("write your pallas kernel here")


# speedup vs baseline: 4.0004x; 4.0004x over previous
import functools

import jax
import jax.numpy as jnp
from jax import lax
from jax.experimental import pallas as pl
from jax.experimental.pallas import tpu as pltpu
from jax.experimental.pallas import tpu_sc as plsc

N = 10000
E = 320000
IN_FEAT = 128
OUT_FEAT = 128
HID = 256
NEG_SLOPE = 0.01
EPS = 1e-5

F = 128
L = 16
NC = 2
NS = 16
NW = NC * NS
EB = E // NW
SUB = 25
KSUB = 8
CHUNK = SUB * KSUB
NBLK = EB // CHUNK
ROWS_T = 624
TAIL0 = NS * ROWS_T

BLK = 2000
GRID = N // BLK


def _spmm_body(xw_hbm, src_hbm, dst_hbm, w_hbm, out_hbm,
               src_v, dst_v, w_v, rows_v, acc_sh, sem):
    c = lax.axis_index("c")
    s = lax.axis_index("s")
    wid = c * NS + s

    zeros = jnp.zeros((L,), jnp.float32)

    def zero_row(e, carry):
        for k in range(F // L):
            rows_v[e, pl.ds(k * L, L)] = zeros
        return carry

    lax.fori_loop(0, CHUNK, zero_row, 0)
    row0 = pl.multiple_of(s * ROWS_T, 8)
    for t in range(ROWS_T // CHUNK):
        pltpu.sync_copy(rows_v, acc_sh.at[pl.ds(row0 + t * CHUNK, CHUNK)])
    _rem = ROWS_T % CHUNK
    if _rem:
        pltpu.sync_copy(
            rows_v.at[pl.ds(0, _rem)],
            acc_sh.at[pl.ds(row0 + (ROWS_T // CHUNK) * CHUNK, _rem)])

    @pl.when(s == NS - 1)
    def _():
        pltpu.sync_copy(rows_v.at[pl.ds(0, N - TAIL0)],
                        acc_sh.at[pl.ds(TAIL0, N - TAIL0)])

    plsc.subcore_barrier()

    def block(b, carry):
        irow = pl.multiple_of((wid * NBLK + b) * KSUB, 8)
        ebase = pl.multiple_of((wid * NBLK + b) * CHUNK, 8)
        pltpu.sync_copy(src_hbm.at[pl.ds(irow, KSUB)], src_v)
        pltpu.sync_copy(dst_hbm.at[pl.ds(irow, KSUB)], dst_v)
        pltpu.sync_copy(w_hbm.at[pl.ds(ebase, CHUNK)], w_v)
        for j in range(KSUB):
            pltpu.async_copy(xw_hbm.at[src_v.at[j]],
                             rows_v.at[pl.ds(j * SUB, SUB)], sem).wait()

        def scale(g, carry2):
            w16 = w_v[pl.ds(g * L, L)]
            for j in range(L):
                e = g * L + j
                w_s = w16[j]
                for k in range(F // L):
                    sl = pl.ds(k * L, L)
                    rows_v[e, sl] = rows_v[e, sl] * w_s
            return carry2

        lax.fori_loop(0, CHUNK // L, scale, 0)
        if CHUNK % L:
            w16 = w_v[pl.ds(CHUNK - L, L)]
            for j in range(L - CHUNK % L, L):
                e = CHUNK - L + j
                w_s = w16[j]
                for k in range(F // L):
                    sl = pl.ds(k * L, L)
                    rows_v[e, sl] = rows_v[e, sl] * w_s
        for j in range(KSUB):
            pltpu.sync_copy(rows_v.at[pl.ds(j * SUB, SUB)],
                            acc_sh.at[dst_v.at[j]], add=True)
        return carry

    lax.fori_loop(0, NBLK, block, 0)
    plsc.subcore_barrier()

    pltpu.sync_copy(acc_sh.at[pl.ds(row0, ROWS_T)],
                    out_hbm.at[c, pl.ds(row0, ROWS_T)])

    @pl.when(s == NS - 1)
    def _():
        pltpu.sync_copy(acc_sh.at[pl.ds(TAIL0, N - TAIL0)],
                        out_hbm.at[c, pl.ds(TAIL0, N - TAIL0)])


_spmm_slab = functools.partial(
    pl.kernel,
    out_type=jax.ShapeDtypeStruct((NC, N, F), jnp.float32),
    mesh=plsc.VectorSubcoreMesh(core_axis_name="c", subcore_axis_name="s",
                                num_cores=NC, num_subcores=NS),
    scratch_types=[
        pltpu.VMEM((KSUB, SUB), jnp.int32),
        pltpu.VMEM((KSUB, SUB), jnp.int32),
        pltpu.VMEM((CHUNK,), jnp.float32),
        pltpu.VMEM((CHUNK, F), jnp.float32),
        pltpu.VMEM_SHARED((N, F), jnp.float32),
        pltpu.SemaphoreType.DMA,
    ],
)(_spmm_body)


def _leaky(v):
    return jnp.where(v >= 0, v, NEG_SLOPE * v)


def _dot(a, b):
    return jnp.dot(a, b, preferred_element_type=jnp.float32)


def _stage_a_body(x_ref, wemb_ref, bemb_ref, t1_ref, h_ref, a_ref, b_ref):
    h = _leaky(_dot(x_ref[...], wemb_ref[...]) + bemb_ref[...])
    h_ref[...] = h
    ht1 = _dot(h, t1_ref[...])
    a_ref[...] = ht1[:, :F]
    b_ref[...] = ht1[:, F:]


def _stage_b1_body(h_ref, pa_ref, pb_ref, t0_ref, b_ref, y_ref, st_ref):
    i = pl.program_id(0)
    s1 = jnp.concatenate([pa_ref[0] + pa_ref[1], pb_ref[0] + pb_ref[1]],
                         axis=1)
    y = _dot(h_ref[...], t0_ref[...]) + s1 + b_ref[...]
    y_ref[...] = y
    part = jnp.concatenate(
        [jnp.sum(y, 0, keepdims=True), jnp.sum(y * y, 0, keepdims=True)], 0)

    @pl.when(i == 0)
    def _():
        st_ref[...] = part

    @pl.when(i > 0)
    def _():
        st_ref[...] = st_ref[...] + part


def _stage_b2_body(y_ref, st_ref, g_ref, be_ref, h_ref, t1_ref,
                   h1_ref, a_ref, b_ref):
    m = st_ref[0:1] * (1.0 / N)
    v = st_ref[1:2] * (1.0 / N) - m * m
    yn = (y_ref[...] - m) / jnp.sqrt(v + EPS) * g_ref[...] + be_ref[...]
    h1 = _leaky(yn)
    h1_ref[...] = h1
    ht1 = _dot(h_ref[...], t1_ref[0:HID]) + _dot(h1, t1_ref[HID:2 * HID])
    a_ref[...] = ht1[:, :F]
    b_ref[...] = ht1[:, F:]


def _stage_c1_body(h_ref, h1_ref, pa_ref, pb_ref, t0_ref, b_ref,
                   y_ref, st_ref):
    i = pl.program_id(0)
    s2 = jnp.concatenate([pa_ref[0] + pa_ref[1], pb_ref[0] + pb_ref[1]],
                         axis=1)
    y = (_dot(h_ref[...], t0_ref[0:HID]) + _dot(h1_ref[...], t0_ref[HID:])
         + s2 + b_ref[...])
    y_ref[...] = y
    part = jnp.concatenate(
        [jnp.sum(y, 0, keepdims=True), jnp.sum(y * y, 0, keepdims=True)], 0)

    @pl.when(i == 0)
    def _():
        st_ref[...] = part

    @pl.when(i > 0)
    def _():
        st_ref[...] = st_ref[...] + part


def _stage_c2_body(y_ref, st_ref, g_ref, be_ref, h_ref, h1_ref, t1_ref,
                   h2_ref, ht1_ref):
    m = st_ref[0:1] * (1.0 / N)
    v = st_ref[1:2] * (1.0 / N) - m * m
    yn = (y_ref[...] - m) / jnp.sqrt(v + EPS) * g_ref[...] + be_ref[...]
    h2 = _leaky(yn)
    h2_ref[...] = h2
    ht1_ref[...] = (_dot(h_ref[...], t1_ref[0:HID])
                    + _dot(h1_ref[...], t1_ref[HID:2 * HID])
                    + _dot(h2, t1_ref[2 * HID:]))


def _stage_d_body(h_ref, h1_ref, h2_ref, p_ref, t0_ref, out_ref):
    out_ref[...] = (_dot(h_ref[...], t0_ref[0:HID])
                    + _dot(h1_ref[...], t0_ref[HID:2 * HID])
                    + _dot(h2_ref[...], t0_ref[2 * HID:])
                    + p_ref[0] + p_ref[1])


def _row_spec(width):
    return pl.BlockSpec((BLK, width), lambda i: (i, 0))


def _part_spec():
    return pl.BlockSpec((NC, BLK, F), lambda i: (0, i, 0))


def _full_spec(shape):
    return pl.BlockSpec(shape, lambda i: tuple(0 for _ in shape))


def _stats_spec():
    return pl.BlockSpec((2, HID), lambda i: (0, 0))


def kernel(x, edge_index, edge_weight, g_size, W_emb, b_emb, gc1_T0, gc1_T1,
           gc1_b, gc1_gamma, gc1_beta, gc2_T0, gc2_T1, gc2_b, gc2_gamma,
           gc2_beta, gcl_T0, gcl_T1):
    src2d = edge_index[0].reshape(E // SUB, SUB)
    dst2d = edge_index[1].reshape(E // SUB, SUB)
    f32 = jnp.float32

    h, h1t1a, h1t1b = pl.pallas_call(
        _stage_a_body,
        grid=(GRID,),
        in_specs=[_row_spec(IN_FEAT), _full_spec((IN_FEAT, HID)),
                  _full_spec((1, HID)), _full_spec((HID, HID))],
        out_specs=[_row_spec(HID), _row_spec(F), _row_spec(F)],
        out_shape=[jax.ShapeDtypeStruct((N, HID), f32),
                   jax.ShapeDtypeStruct((N, F), f32),
                   jax.ShapeDtypeStruct((N, F), f32)],
    )(x, W_emb, b_emb.reshape(1, HID), gc1_T1)

    p1a = _spmm_slab(h1t1a, src2d, dst2d, edge_weight)
    p1b = _spmm_slab(h1t1b, src2d, dst2d, edge_weight)

    y1, st1 = pl.pallas_call(
        _stage_b1_body,
        grid=(GRID,),
        in_specs=[_row_spec(HID), _part_spec(), _part_spec(),
                  _full_spec((HID, HID)), _full_spec((1, HID))],
        out_specs=[_row_spec(HID), _stats_spec()],
        out_shape=[jax.ShapeDtypeStruct((N, HID), f32),
                   jax.ShapeDtypeStruct((2, HID), f32)],
    )(h, p1a, p1b, gc1_T0, gc1_b.reshape(1, HID))

    h1, h2t1a, h2t1b = pl.pallas_call(
        _stage_b2_body,
        grid=(GRID,),
        in_specs=[_row_spec(HID), _stats_spec(), _full_spec((1, HID)),
                  _full_spec((1, HID)), _row_spec(HID),
                  _full_spec((2 * HID, HID))],
        out_specs=[_row_spec(HID), _row_spec(F), _row_spec(F)],
        out_shape=[jax.ShapeDtypeStruct((N, HID), f32),
                   jax.ShapeDtypeStruct((N, F), f32),
                   jax.ShapeDtypeStruct((N, F), f32)],
    )(y1, st1, gc1_gamma.reshape(1, HID), gc1_beta.reshape(1, HID), h,
      gc2_T1)

    p2a = _spmm_slab(h2t1a, src2d, dst2d, edge_weight)
    p2b = _spmm_slab(h2t1b, src2d, dst2d, edge_weight)

    y2, st2 = pl.pallas_call(
        _stage_c1_body,
        grid=(GRID,),
        in_specs=[_row_spec(HID), _row_spec(HID), _part_spec(), _part_spec(),
                  _full_spec((2 * HID, HID)), _full_spec((1, HID))],
        out_specs=[_row_spec(HID), _stats_spec()],
        out_shape=[jax.ShapeDtypeStruct((N, HID), f32),
                   jax.ShapeDtypeStruct((2, HID), f32)],
    )(h, h1, p2a, p2b, gc2_T0, gc2_b.reshape(1, HID))

    h2, h3t1 = pl.pallas_call(
        _stage_c2_body,
        grid=(GRID,),
        in_specs=[_row_spec(HID), _stats_spec(), _full_spec((1, HID)),
                  _full_spec((1, HID)), _row_spec(HID), _row_spec(HID),
                  _full_spec((3 * HID, OUT_FEAT))],
        out_specs=[_row_spec(HID), _row_spec(OUT_FEAT)],
        out_shape=[jax.ShapeDtypeStruct((N, HID), f32),
                   jax.ShapeDtypeStruct((N, OUT_FEAT), f32)],
    )(y2, st2, gc2_gamma.reshape(1, HID), gc2_beta.reshape(1, HID), h, h1,
      gcl_T1)

    p3 = _spmm_slab(h3t1, src2d, dst2d, edge_weight)

    out = pl.pallas_call(
        _stage_d_body,
        grid=(GRID,),
        in_specs=[_row_spec(HID), _row_spec(HID), _row_spec(HID),
                  _part_spec(), _full_spec((3 * HID, OUT_FEAT))],
        out_specs=_row_spec(OUT_FEAT),
        out_shape=jax.ShapeDtypeStruct((N, OUT_FEAT), f32),
    )(h, h1, h2, p3, gcl_T0)

    return out


# speedup vs baseline: 7.0147x; 1.7535x over previous
import functools

import jax
import jax.numpy as jnp
from jax import lax
from jax.experimental import pallas as pl
from jax.experimental.pallas import tpu as pltpu
from jax.experimental.pallas import tpu_sc as plsc

N = 10000
E = 320000
IN_FEAT = 128
OUT_FEAT = 128
HID = 256
NEG_SLOPE = 0.01
EPS = 1e-5

F = 128
L = 16
NC = 2
NS = 16
NW = NC * NS
EB = E // NW
SUB = 25
KSUB = 8
CHUNK = SUB * KSUB
NBLK = EB // CHUNK
ROWS_T = 624
TAIL0 = NS * ROWS_T

BLK = 2000
GRID = N // BLK


def _spmm_body(xw_hbm, src_hbm, dst_hbm, w_hbm, out_hbm,
               src_v, dst_v, w_v, rows_v, acc_sh, sem):
    c = lax.axis_index("c")
    s = lax.axis_index("s")
    wid = c * NS + s

    zeros = jnp.zeros((L,), jnp.float32)

    def zero_row(e, carry):
        for k in range(F // L):
            rows_v[e, pl.ds(k * L, L)] = zeros
        return carry

    lax.fori_loop(0, CHUNK, zero_row, 0)
    row0 = pl.multiple_of(s * ROWS_T, 8)
    for t in range(ROWS_T // CHUNK):
        pltpu.sync_copy(rows_v, acc_sh.at[pl.ds(row0 + t * CHUNK, CHUNK)])
    _rem = ROWS_T % CHUNK
    if _rem:
        pltpu.sync_copy(
            rows_v.at[pl.ds(0, _rem)],
            acc_sh.at[pl.ds(row0 + (ROWS_T // CHUNK) * CHUNK, _rem)])

    @pl.when(s == NS - 1)
    def _():
        pltpu.sync_copy(rows_v.at[pl.ds(0, N - TAIL0)],
                        acc_sh.at[pl.ds(TAIL0, N - TAIL0)])

    plsc.subcore_barrier()

    def block(b, carry):
        irow = pl.multiple_of((wid * NBLK + b) * KSUB, 8)
        ebase = pl.multiple_of((wid * NBLK + b) * CHUNK, 8)
        pltpu.sync_copy(src_hbm.at[pl.ds(irow, KSUB)], src_v)
        pltpu.sync_copy(dst_hbm.at[pl.ds(irow, KSUB)], dst_v)
        pltpu.sync_copy(w_hbm.at[pl.ds(ebase, CHUNK)], w_v)
        descs = [
            pltpu.async_copy(xw_hbm.at[src_v.at[j]],
                             rows_v.at[pl.ds(j * SUB, SUB)], sem)
            for j in range(KSUB)
        ]
        for d in descs:
            d.wait()

        def scale(g, carry2):
            w16 = w_v[pl.ds(g * L, L)]
            for j in range(L):
                e = g * L + j
                w_s = w16[j]
                for k in range(F // L):
                    sl = pl.ds(k * L, L)
                    rows_v[e, sl] = rows_v[e, sl] * w_s
            return carry2

        lax.fori_loop(0, CHUNK // L, scale, 0)
        if CHUNK % L:
            w16 = w_v[pl.ds(CHUNK - L, L)]
            for j in range(L - CHUNK % L, L):
                e = CHUNK - L + j
                w_s = w16[j]
                for k in range(F // L):
                    sl = pl.ds(k * L, L)
                    rows_v[e, sl] = rows_v[e, sl] * w_s
        sdescs = [
            pltpu.async_copy(rows_v.at[pl.ds(j * SUB, SUB)],
                             acc_sh.at[dst_v.at[j]], sem, add=True)
            for j in range(KSUB)
        ]
        for d in sdescs:
            d.wait()
        return carry

    lax.fori_loop(0, NBLK, block, 0)
    plsc.subcore_barrier()

    pltpu.sync_copy(acc_sh.at[pl.ds(row0, ROWS_T)],
                    out_hbm.at[c, pl.ds(row0, ROWS_T)])

    @pl.when(s == NS - 1)
    def _():
        pltpu.sync_copy(acc_sh.at[pl.ds(TAIL0, N - TAIL0)],
                        out_hbm.at[c, pl.ds(TAIL0, N - TAIL0)])


_spmm_slab = functools.partial(
    pl.kernel,
    out_type=jax.ShapeDtypeStruct((NC, N, F), jnp.float32),
    mesh=plsc.VectorSubcoreMesh(core_axis_name="c", subcore_axis_name="s",
                                num_cores=NC, num_subcores=NS),
    scratch_types=[
        pltpu.VMEM((KSUB, SUB), jnp.int32),
        pltpu.VMEM((KSUB, SUB), jnp.int32),
        pltpu.VMEM((CHUNK,), jnp.float32),
        pltpu.VMEM((CHUNK, F), jnp.float32),
        pltpu.VMEM_SHARED((N, F), jnp.float32),
        pltpu.SemaphoreType.DMA,
    ],
)(_spmm_body)


def _leaky(v):
    return jnp.where(v >= 0, v, NEG_SLOPE * v)


def _dot(a, b):
    return jnp.dot(a, b, preferred_element_type=jnp.float32)


def _stage_a_body(x_ref, wemb_ref, bemb_ref, t1_ref, h_ref, a_ref, b_ref):
    h = _leaky(_dot(x_ref[...], wemb_ref[...]) + bemb_ref[...])
    h_ref[...] = h
    ht1 = _dot(h, t1_ref[...])
    a_ref[...] = ht1[:, :F]
    b_ref[...] = ht1[:, F:]


def _stage_b1_body(h_ref, pa_ref, pb_ref, t0_ref, b_ref, y_ref, st_ref):
    i = pl.program_id(0)
    s1 = jnp.concatenate([pa_ref[0] + pa_ref[1], pb_ref[0] + pb_ref[1]],
                         axis=1)
    y = _dot(h_ref[...], t0_ref[...]) + s1 + b_ref[...]
    y_ref[...] = y
    part = jnp.concatenate(
        [jnp.sum(y, 0, keepdims=True), jnp.sum(y * y, 0, keepdims=True)], 0)

    @pl.when(i == 0)
    def _():
        st_ref[...] = part

    @pl.when(i > 0)
    def _():
        st_ref[...] = st_ref[...] + part


def _stage_b2_body(y_ref, st_ref, g_ref, be_ref, h_ref, t1_ref,
                   h1_ref, a_ref, b_ref):
    m = st_ref[0:1] * (1.0 / N)
    v = st_ref[1:2] * (1.0 / N) - m * m
    yn = (y_ref[...] - m) / jnp.sqrt(v + EPS) * g_ref[...] + be_ref[...]
    h1 = _leaky(yn)
    h1_ref[...] = h1
    ht1 = _dot(h_ref[...], t1_ref[0:HID]) + _dot(h1, t1_ref[HID:2 * HID])
    a_ref[...] = ht1[:, :F]
    b_ref[...] = ht1[:, F:]


def _stage_c1_body(h_ref, h1_ref, pa_ref, pb_ref, t0_ref, b_ref,
                   y_ref, st_ref):
    i = pl.program_id(0)
    s2 = jnp.concatenate([pa_ref[0] + pa_ref[1], pb_ref[0] + pb_ref[1]],
                         axis=1)
    y = (_dot(h_ref[...], t0_ref[0:HID]) + _dot(h1_ref[...], t0_ref[HID:])
         + s2 + b_ref[...])
    y_ref[...] = y
    part = jnp.concatenate(
        [jnp.sum(y, 0, keepdims=True), jnp.sum(y * y, 0, keepdims=True)], 0)

    @pl.when(i == 0)
    def _():
        st_ref[...] = part

    @pl.when(i > 0)
    def _():
        st_ref[...] = st_ref[...] + part


def _stage_c2_body(y_ref, st_ref, g_ref, be_ref, h_ref, h1_ref, t1_ref,
                   h2_ref, ht1_ref):
    m = st_ref[0:1] * (1.0 / N)
    v = st_ref[1:2] * (1.0 / N) - m * m
    yn = (y_ref[...] - m) / jnp.sqrt(v + EPS) * g_ref[...] + be_ref[...]
    h2 = _leaky(yn)
    h2_ref[...] = h2
    ht1_ref[...] = (_dot(h_ref[...], t1_ref[0:HID])
                    + _dot(h1_ref[...], t1_ref[HID:2 * HID])
                    + _dot(h2, t1_ref[2 * HID:]))


def _stage_d_body(h_ref, h1_ref, h2_ref, p_ref, t0_ref, out_ref):
    out_ref[...] = (_dot(h_ref[...], t0_ref[0:HID])
                    + _dot(h1_ref[...], t0_ref[HID:2 * HID])
                    + _dot(h2_ref[...], t0_ref[2 * HID:])
                    + p_ref[0] + p_ref[1])


def _row_spec(width):
    return pl.BlockSpec((BLK, width), lambda i: (i, 0))


def _part_spec():
    return pl.BlockSpec((NC, BLK, F), lambda i: (0, i, 0))


def _full_spec(shape):
    return pl.BlockSpec(shape, lambda i: tuple(0 for _ in shape))


def _stats_spec():
    return pl.BlockSpec((2, HID), lambda i: (0, 0))


def kernel(x, edge_index, edge_weight, g_size, W_emb, b_emb, gc1_T0, gc1_T1,
           gc1_b, gc1_gamma, gc1_beta, gc2_T0, gc2_T1, gc2_b, gc2_gamma,
           gc2_beta, gcl_T0, gcl_T1):
    src2d = edge_index[0].reshape(E // SUB, SUB)
    dst2d = edge_index[1].reshape(E // SUB, SUB)
    f32 = jnp.float32

    h, h1t1a, h1t1b = pl.pallas_call(
        _stage_a_body,
        grid=(GRID,),
        in_specs=[_row_spec(IN_FEAT), _full_spec((IN_FEAT, HID)),
                  _full_spec((1, HID)), _full_spec((HID, HID))],
        out_specs=[_row_spec(HID), _row_spec(F), _row_spec(F)],
        out_shape=[jax.ShapeDtypeStruct((N, HID), f32),
                   jax.ShapeDtypeStruct((N, F), f32),
                   jax.ShapeDtypeStruct((N, F), f32)],
    )(x, W_emb, b_emb.reshape(1, HID), gc1_T1)

    p1a = _spmm_slab(h1t1a, src2d, dst2d, edge_weight)
    p1b = _spmm_slab(h1t1b, src2d, dst2d, edge_weight)

    y1, st1 = pl.pallas_call(
        _stage_b1_body,
        grid=(GRID,),
        in_specs=[_row_spec(HID), _part_spec(), _part_spec(),
                  _full_spec((HID, HID)), _full_spec((1, HID))],
        out_specs=[_row_spec(HID), _stats_spec()],
        out_shape=[jax.ShapeDtypeStruct((N, HID), f32),
                   jax.ShapeDtypeStruct((2, HID), f32)],
    )(h, p1a, p1b, gc1_T0, gc1_b.reshape(1, HID))

    h1, h2t1a, h2t1b = pl.pallas_call(
        _stage_b2_body,
        grid=(GRID,),
        in_specs=[_row_spec(HID), _stats_spec(), _full_spec((1, HID)),
                  _full_spec((1, HID)), _row_spec(HID),
                  _full_spec((2 * HID, HID))],
        out_specs=[_row_spec(HID), _row_spec(F), _row_spec(F)],
        out_shape=[jax.ShapeDtypeStruct((N, HID), f32),
                   jax.ShapeDtypeStruct((N, F), f32),
                   jax.ShapeDtypeStruct((N, F), f32)],
    )(y1, st1, gc1_gamma.reshape(1, HID), gc1_beta.reshape(1, HID), h,
      gc2_T1)

    p2a = _spmm_slab(h2t1a, src2d, dst2d, edge_weight)
    p2b = _spmm_slab(h2t1b, src2d, dst2d, edge_weight)

    y2, st2 = pl.pallas_call(
        _stage_c1_body,
        grid=(GRID,),
        in_specs=[_row_spec(HID), _row_spec(HID), _part_spec(), _part_spec(),
                  _full_spec((2 * HID, HID)), _full_spec((1, HID))],
        out_specs=[_row_spec(HID), _stats_spec()],
        out_shape=[jax.ShapeDtypeStruct((N, HID), f32),
                   jax.ShapeDtypeStruct((2, HID), f32)],
    )(h, h1, p2a, p2b, gc2_T0, gc2_b.reshape(1, HID))

    h2, h3t1 = pl.pallas_call(
        _stage_c2_body,
        grid=(GRID,),
        in_specs=[_row_spec(HID), _stats_spec(), _full_spec((1, HID)),
                  _full_spec((1, HID)), _row_spec(HID), _row_spec(HID),
                  _full_spec((3 * HID, OUT_FEAT))],
        out_specs=[_row_spec(HID), _row_spec(OUT_FEAT)],
        out_shape=[jax.ShapeDtypeStruct((N, HID), f32),
                   jax.ShapeDtypeStruct((N, OUT_FEAT), f32)],
    )(y2, st2, gc2_gamma.reshape(1, HID), gc2_beta.reshape(1, HID), h, h1,
      gcl_T1)

    p3 = _spmm_slab(h3t1, src2d, dst2d, edge_weight)

    out = pl.pallas_call(
        _stage_d_body,
        grid=(GRID,),
        in_specs=[_row_spec(HID), _row_spec(HID), _row_spec(HID),
                  _part_spec(), _full_spec((3 * HID, OUT_FEAT))],
        out_specs=_row_spec(OUT_FEAT),
        out_shape=jax.ShapeDtypeStruct((N, OUT_FEAT), f32),
    )(h, h1, h2, p3, gcl_T0)

    return out


# speedup vs baseline: 12.0052x; 1.7114x over previous
import functools

import jax
import jax.numpy as jnp
from jax import lax
from jax.experimental import pallas as pl
from jax.experimental.pallas import tpu as pltpu
from jax.experimental.pallas import tpu_sc as plsc

N = 10000
E = 320000
IN_FEAT = 128
OUT_FEAT = 128
HID = 256
NEG_SLOPE = 0.01
EPS = 1e-5

F = 128
L = 16
NC = 2
NS = 16
NW = NC * NS
EB = E // NW
SUB = 25
KSUB = 8
CHUNK = SUB * KSUB
NBLK = EB // CHUNK
SB = 5
NSB = NBLK // SB
H = CHUNK // 2
KH = KSUB // 2
ROWS_T = 624
TAIL0 = NS * ROWS_T

BLK = 2000
GRID = N // BLK


NT = NBLK * KSUB
RING = KSUB
SLOTR = 32
ZROWS = RING * SLOTR
LEAD = 6
IBUF = 4


def _spmm_body(xw_hbm, src_hbm, dst_hbm, w_hbm, out_hbm,
               src_v, dst_v, w_v, rows_v, acc_sh, sem_g, sem_s, sem_i):
    c = lax.axis_index("c")
    s = lax.axis_index("s")
    wid = c * NS + s

    def fire_idx(b, pset):
        r0 = pl.multiple_of((wid * NBLK + b) * KSUB, 8)
        e0 = pl.multiple_of((wid * NBLK + b) * CHUNK, 8)
        pltpu.async_copy(src_hbm.at[pl.ds(r0, KSUB)], src_v.at[pset], sem_i)
        pltpu.async_copy(dst_hbm.at[pl.ds(r0, KSUB)], dst_v.at[pset], sem_i)
        pltpu.async_copy(
            w_hbm.at[pl.ds(e0, CHUNK)],
            w_v.at[pl.ds(pl.multiple_of(pset * CHUNK, 8), CHUNK)], sem_i)

    def drain_idx():
        pltpu.make_async_copy(src_hbm.at[pl.ds(0, KSUB)], src_v.at[0],
                              sem_i).wait()
        pltpu.make_async_copy(dst_hbm.at[pl.ds(0, KSUB)], dst_v.at[0],
                              sem_i).wait()
        pltpu.make_async_copy(w_hbm.at[pl.ds(0, CHUNK)],
                              w_v.at[pl.ds(0, CHUNK)], sem_i).wait()

    def fire_gather(pset, j, slot):
        pltpu.async_copy(xw_hbm.at[src_v.at[pset, j]],
                         rows_v.at[pl.ds(slot * SLOTR, SUB)], sem_g)

    def drain_gather():
        pltpu.make_async_copy(xw_hbm.at[src_v.at[0, 0]],
                              rows_v.at[pl.ds(0, SUB)], sem_g).wait()

    def fire_scatter(pset, j, slot):
        pltpu.async_copy(rows_v.at[pl.ds(slot * SLOTR, SUB)],
                         acc_sh.at[dst_v.at[pset, j]], sem_s, add=True)

    def drain_scatter():
        pltpu.make_async_copy(rows_v.at[pl.ds(0, SUB)],
                              acc_sh.at[dst_v.at[0, 0]], sem_s).wait()

    def scale_sub(pset, j, slot):
        base_w = pset * CHUNK + j * SUB
        base_e = slot * SLOTR
        w16 = w_v[pl.ds(base_w, L)]
        for i in range(L):
            w_s = w16[i]
            e = base_e + i
            for k in range(F // L):
                sl = pl.ds(k * L, L)
                rows_v[e, sl] = rows_v[e, sl] * w_s
        w16 = w_v[pl.ds(base_w + SUB - L, L)]
        for i in range(2 * L - SUB, L):
            w_s = w16[i]
            e = base_e + (SUB - L) + i
            for k in range(F // L):
                sl = pl.ds(k * L, L)
                rows_v[e, sl] = rows_v[e, sl] * w_s

    fire_idx(0, 0)
    fire_idx(1, 1)
    zeros = jnp.zeros((L,), jnp.float32)

    def zero_row(e, carry):
        for k in range(F // L):
            rows_v[e, pl.ds(k * L, L)] = zeros
        return carry

    lax.fori_loop(0, ZROWS, zero_row, 0)
    row0 = pl.multiple_of(s * ROWS_T, 8)
    for t in range(ROWS_T // ZROWS):
        pltpu.sync_copy(rows_v, acc_sh.at[pl.ds(row0 + t * ZROWS, ZROWS)])
    _rem = ROWS_T % ZROWS
    if _rem:
        pltpu.sync_copy(
            rows_v.at[pl.ds(0, _rem)],
            acc_sh.at[pl.ds(row0 + (ROWS_T // ZROWS) * ZROWS, _rem)])

    @pl.when(s == NS - 1)
    def _():
        pltpu.sync_copy(rows_v.at[pl.ds(0, N - TAIL0)],
                        acc_sh.at[pl.ds(TAIL0, N - TAIL0)])

    plsc.subcore_barrier()

    drain_idx()
    for j in range(LEAD):
        fire_gather(0, j, j)

    def block(b, carry):
        pb = b % IBUF

        @pl.when(b + 2 < NBLK)
        def _():
            fire_idx(b + 2, (b + 2) % IBUF)

        for j in range(KSUB):
            drain_gather()
            scale_sub(pb, j, j)
            fire_scatter(pb, j, j)
            if j == 1:
                @pl.when(b < NBLK - 1)
                def _():
                    drain_idx()
            if j < RING - LEAD:
                @pl.when(b > 0)
                def _():
                    drain_scatter()
            else:
                drain_scatter()
            jn = j + LEAD
            if jn < KSUB:
                fire_gather(pb, jn, jn)
            else:
                @pl.when(b < NBLK - 1)
                def _():
                    fire_gather((b + 1) % IBUF, jn - KSUB, jn - KSUB)
        return carry

    lax.fori_loop(0, NBLK, block, 0)
    for _ in range(RING - LEAD):
        drain_scatter()
    plsc.subcore_barrier()

    pltpu.sync_copy(acc_sh.at[pl.ds(row0, ROWS_T)],
                    out_hbm.at[c, pl.ds(row0, ROWS_T)])

    @pl.when(s == NS - 1)
    def _():
        pltpu.sync_copy(acc_sh.at[pl.ds(TAIL0, N - TAIL0)],
                        out_hbm.at[c, pl.ds(TAIL0, N - TAIL0)])


_spmm_slab = functools.partial(
    pl.kernel,
    out_type=jax.ShapeDtypeStruct((NC, N, F), jnp.float32),
    mesh=plsc.VectorSubcoreMesh(core_axis_name="c", subcore_axis_name="s",
                                num_cores=NC, num_subcores=NS),
    scratch_types=[
        pltpu.VMEM((IBUF, KSUB, SUB), jnp.int32),
        pltpu.VMEM((IBUF, KSUB, SUB), jnp.int32),
        pltpu.VMEM((IBUF * CHUNK,), jnp.float32),
        pltpu.VMEM((ZROWS, F), jnp.float32),
        pltpu.VMEM_SHARED((N, F), jnp.float32),
        pltpu.SemaphoreType.DMA,
        pltpu.SemaphoreType.DMA,
        pltpu.SemaphoreType.DMA,
    ],
)(_spmm_body)


def _leaky(v):
    return jnp.where(v >= 0, v, NEG_SLOPE * v)


def _dot(a, b):
    return jnp.dot(a, b, preferred_element_type=jnp.float32)


def _stage_a_body(x_ref, wemb_ref, bemb_ref, t1_ref, h_ref, a_ref, b_ref):
    h = _leaky(_dot(x_ref[...], wemb_ref[...]) + bemb_ref[...])
    h_ref[...] = h
    ht1 = _dot(h, t1_ref[...])
    a_ref[...] = ht1[:, :F]
    b_ref[...] = ht1[:, F:]


def _stage_b1_body(h_ref, pa_ref, pb_ref, t0_ref, b_ref, y_ref, st_ref):
    i = pl.program_id(0)
    s1 = jnp.concatenate([pa_ref[0] + pa_ref[1], pb_ref[0] + pb_ref[1]],
                         axis=1)
    y = _dot(h_ref[...], t0_ref[...]) + s1 + b_ref[...]
    y_ref[...] = y
    part = jnp.concatenate(
        [jnp.sum(y, 0, keepdims=True), jnp.sum(y * y, 0, keepdims=True)], 0)

    @pl.when(i == 0)
    def _():
        st_ref[...] = part

    @pl.when(i > 0)
    def _():
        st_ref[...] = st_ref[...] + part


def _stage_b2_body(y_ref, st_ref, g_ref, be_ref, h_ref, t1_ref,
                   h1_ref, a_ref, b_ref):
    m = st_ref[0:1] * (1.0 / N)
    v = st_ref[1:2] * (1.0 / N) - m * m
    yn = (y_ref[...] - m) / jnp.sqrt(v + EPS) * g_ref[...] + be_ref[...]
    h1 = _leaky(yn)
    h1_ref[...] = h1
    ht1 = _dot(h_ref[...], t1_ref[0:HID]) + _dot(h1, t1_ref[HID:2 * HID])
    a_ref[...] = ht1[:, :F]
    b_ref[...] = ht1[:, F:]


def _stage_c1_body(h_ref, h1_ref, pa_ref, pb_ref, t0_ref, b_ref,
                   y_ref, st_ref):
    i = pl.program_id(0)
    s2 = jnp.concatenate([pa_ref[0] + pa_ref[1], pb_ref[0] + pb_ref[1]],
                         axis=1)
    y = (_dot(h_ref[...], t0_ref[0:HID]) + _dot(h1_ref[...], t0_ref[HID:])
         + s2 + b_ref[...])
    y_ref[...] = y
    part = jnp.concatenate(
        [jnp.sum(y, 0, keepdims=True), jnp.sum(y * y, 0, keepdims=True)], 0)

    @pl.when(i == 0)
    def _():
        st_ref[...] = part

    @pl.when(i > 0)
    def _():
        st_ref[...] = st_ref[...] + part


def _stage_c2_body(y_ref, st_ref, g_ref, be_ref, h_ref, h1_ref, t1_ref,
                   h2_ref, ht1_ref):
    m = st_ref[0:1] * (1.0 / N)
    v = st_ref[1:2] * (1.0 / N) - m * m
    yn = (y_ref[...] - m) / jnp.sqrt(v + EPS) * g_ref[...] + be_ref[...]
    h2 = _leaky(yn)
    h2_ref[...] = h2
    ht1_ref[...] = (_dot(h_ref[...], t1_ref[0:HID])
                    + _dot(h1_ref[...], t1_ref[HID:2 * HID])
                    + _dot(h2, t1_ref[2 * HID:]))


def _stage_d_body(h_ref, h1_ref, h2_ref, p_ref, t0_ref, out_ref):
    out_ref[...] = (_dot(h_ref[...], t0_ref[0:HID])
                    + _dot(h1_ref[...], t0_ref[HID:2 * HID])
                    + _dot(h2_ref[...], t0_ref[2 * HID:])
                    + p_ref[0] + p_ref[1])


def _row_spec(width):
    return pl.BlockSpec((BLK, width), lambda i: (i, 0))


def _part_spec():
    return pl.BlockSpec((NC, BLK, F), lambda i: (0, i, 0))


def _full_spec(shape):
    return pl.BlockSpec(shape, lambda i: tuple(0 for _ in shape))


def _stats_spec():
    return pl.BlockSpec((2, HID), lambda i: (0, 0))


def kernel(x, edge_index, edge_weight, g_size, W_emb, b_emb, gc1_T0, gc1_T1,
           gc1_b, gc1_gamma, gc1_beta, gc2_T0, gc2_T1, gc2_b, gc2_gamma,
           gc2_beta, gcl_T0, gcl_T1):
    src2d = edge_index[0].reshape(E // SUB, SUB)
    dst2d = edge_index[1].reshape(E // SUB, SUB)
    f32 = jnp.float32

    h, h1t1a, h1t1b = pl.pallas_call(
        _stage_a_body,
        grid=(GRID,),
        in_specs=[_row_spec(IN_FEAT), _full_spec((IN_FEAT, HID)),
                  _full_spec((1, HID)), _full_spec((HID, HID))],
        out_specs=[_row_spec(HID), _row_spec(F), _row_spec(F)],
        out_shape=[jax.ShapeDtypeStruct((N, HID), f32),
                   jax.ShapeDtypeStruct((N, F), f32),
                   jax.ShapeDtypeStruct((N, F), f32)],
    )(x, W_emb, b_emb.reshape(1, HID), gc1_T1)

    p1a = _spmm_slab(h1t1a, src2d, dst2d, edge_weight)
    p1b = _spmm_slab(h1t1b, src2d, dst2d, edge_weight)

    y1, st1 = pl.pallas_call(
        _stage_b1_body,
        grid=(GRID,),
        in_specs=[_row_spec(HID), _part_spec(), _part_spec(),
                  _full_spec((HID, HID)), _full_spec((1, HID))],
        out_specs=[_row_spec(HID), _stats_spec()],
        out_shape=[jax.ShapeDtypeStruct((N, HID), f32),
                   jax.ShapeDtypeStruct((2, HID), f32)],
    )(h, p1a, p1b, gc1_T0, gc1_b.reshape(1, HID))

    h1, h2t1a, h2t1b = pl.pallas_call(
        _stage_b2_body,
        grid=(GRID,),
        in_specs=[_row_spec(HID), _stats_spec(), _full_spec((1, HID)),
                  _full_spec((1, HID)), _row_spec(HID),
                  _full_spec((2 * HID, HID))],
        out_specs=[_row_spec(HID), _row_spec(F), _row_spec(F)],
        out_shape=[jax.ShapeDtypeStruct((N, HID), f32),
                   jax.ShapeDtypeStruct((N, F), f32),
                   jax.ShapeDtypeStruct((N, F), f32)],
    )(y1, st1, gc1_gamma.reshape(1, HID), gc1_beta.reshape(1, HID), h,
      gc2_T1)

    p2a = _spmm_slab(h2t1a, src2d, dst2d, edge_weight)
    p2b = _spmm_slab(h2t1b, src2d, dst2d, edge_weight)

    y2, st2 = pl.pallas_call(
        _stage_c1_body,
        grid=(GRID,),
        in_specs=[_row_spec(HID), _row_spec(HID), _part_spec(), _part_spec(),
                  _full_spec((2 * HID, HID)), _full_spec((1, HID))],
        out_specs=[_row_spec(HID), _stats_spec()],
        out_shape=[jax.ShapeDtypeStruct((N, HID), f32),
                   jax.ShapeDtypeStruct((2, HID), f32)],
    )(h, h1, p2a, p2b, gc2_T0, gc2_b.reshape(1, HID))

    h2, h3t1 = pl.pallas_call(
        _stage_c2_body,
        grid=(GRID,),
        in_specs=[_row_spec(HID), _stats_spec(), _full_spec((1, HID)),
                  _full_spec((1, HID)), _row_spec(HID), _row_spec(HID),
                  _full_spec((3 * HID, OUT_FEAT))],
        out_specs=[_row_spec(HID), _row_spec(OUT_FEAT)],
        out_shape=[jax.ShapeDtypeStruct((N, HID), f32),
                   jax.ShapeDtypeStruct((N, OUT_FEAT), f32)],
    )(y2, st2, gc2_gamma.reshape(1, HID), gc2_beta.reshape(1, HID), h, h1,
      gcl_T1)

    p3 = _spmm_slab(h3t1, src2d, dst2d, edge_weight)

    out = pl.pallas_call(
        _stage_d_body,
        grid=(GRID,),
        in_specs=[_row_spec(HID), _row_spec(HID), _row_spec(HID),
                  _part_spec(), _full_spec((3 * HID, OUT_FEAT))],
        out_specs=_row_spec(OUT_FEAT),
        out_shape=jax.ShapeDtypeStruct((N, OUT_FEAT), f32),
    )(h, h1, h2, p3, gcl_T0)

    return out


# speedup vs baseline: 12.8216x; 1.0680x over previous
import functools

import jax
import jax.numpy as jnp
from jax import lax
from jax.experimental import pallas as pl
from jax.experimental.pallas import tpu as pltpu
from jax.experimental.pallas import tpu_sc as plsc

N = 10000
E = 320000
IN_FEAT = 128
OUT_FEAT = 128
HID = 256
NEG_SLOPE = 0.01
EPS = 1e-5

F = 128
L = 16
NC = 2
NS = 16
NW = NC * NS
EB = E // NW
SUB = 25
KSUB = 8
CHUNK = SUB * KSUB
NBLK = EB // CHUNK
SB = 5
NSB = NBLK // SB
H = CHUNK // 2
KH = KSUB // 2
ROWS_T = 624
TAIL0 = NS * ROWS_T

BLK = 2000
GRID = N // BLK


NT = NBLK * KSUB
RING = KSUB
SLOTR = 32
ZROWS = RING * SLOTR
LEAD = 6
IBUF = 4


def _make_spmm_body(dual):
  nblk = E // ((NS if dual else NW) * CHUNK)

  def _spmm_body(xw_hbm, src_hbm, dst_hbm, w_hbm, out_hbm,
                 src_v, dst_v, w_v, rows_v, acc_sh, sem_g, sem_s, sem_i):
    c = lax.axis_index("c")
    s = lax.axis_index("s")
    wid = s if dual else c * NS + s
    xw = xw_hbm.at[c] if dual else xw_hbm

    def fire_idx(b, pset):
        r0 = pl.multiple_of((wid * nblk + b) * KSUB, 8)
        e0 = pl.multiple_of((wid * nblk + b) * CHUNK, 8)
        pltpu.async_copy(src_hbm.at[pl.ds(r0, KSUB)], src_v.at[pset], sem_i)
        pltpu.async_copy(dst_hbm.at[pl.ds(r0, KSUB)], dst_v.at[pset], sem_i)
        pltpu.async_copy(
            w_hbm.at[pl.ds(e0, CHUNK)],
            w_v.at[pl.ds(pl.multiple_of(pset * CHUNK, 8), CHUNK)], sem_i)

    def drain_idx():
        pltpu.make_async_copy(src_hbm.at[pl.ds(0, KSUB)], src_v.at[0],
                              sem_i).wait()
        pltpu.make_async_copy(dst_hbm.at[pl.ds(0, KSUB)], dst_v.at[0],
                              sem_i).wait()
        pltpu.make_async_copy(w_hbm.at[pl.ds(0, CHUNK)],
                              w_v.at[pl.ds(0, CHUNK)], sem_i).wait()

    def fire_gather(pset, j, slot):
        pltpu.async_copy(xw.at[src_v.at[pset, j]],
                         rows_v.at[pl.ds(slot * SLOTR, SUB)], sem_g)

    def drain_gather():
        pltpu.make_async_copy(xw.at[src_v.at[0, 0]],
                              rows_v.at[pl.ds(0, SUB)], sem_g).wait()

    def fire_scatter(pset, j, slot):
        pltpu.async_copy(rows_v.at[pl.ds(slot * SLOTR, SUB)],
                         acc_sh.at[dst_v.at[pset, j]], sem_s, add=True)

    def drain_scatter():
        pltpu.make_async_copy(rows_v.at[pl.ds(0, SUB)],
                              acc_sh.at[dst_v.at[0, 0]], sem_s).wait()

    def scale_sub(pset, j, slot):
        base_w = pset * CHUNK + j * SUB
        base_e = slot * SLOTR
        w16 = w_v[pl.ds(base_w, L)]
        for i in range(L):
            w_s = w16[i]
            e = base_e + i
            for k in range(F // L):
                sl = pl.ds(k * L, L)
                rows_v[e, sl] = rows_v[e, sl] * w_s
        w16 = w_v[pl.ds(base_w + SUB - L, L)]
        for i in range(2 * L - SUB, L):
            w_s = w16[i]
            e = base_e + (SUB - L) + i
            for k in range(F // L):
                sl = pl.ds(k * L, L)
                rows_v[e, sl] = rows_v[e, sl] * w_s

    fire_idx(0, 0)
    fire_idx(1, 1)
    zeros = jnp.zeros((L,), jnp.float32)

    def zero_row(e, carry):
        for k in range(F // L):
            rows_v[e, pl.ds(k * L, L)] = zeros
        return carry

    lax.fori_loop(0, ZROWS, zero_row, 0)
    row0 = pl.multiple_of(s * ROWS_T, 8)
    for t in range(ROWS_T // ZROWS):
        pltpu.sync_copy(rows_v, acc_sh.at[pl.ds(row0 + t * ZROWS, ZROWS)])
    _rem = ROWS_T % ZROWS
    if _rem:
        pltpu.sync_copy(
            rows_v.at[pl.ds(0, _rem)],
            acc_sh.at[pl.ds(row0 + (ROWS_T // ZROWS) * ZROWS, _rem)])

    @pl.when(s == NS - 1)
    def _():
        pltpu.sync_copy(rows_v.at[pl.ds(0, N - TAIL0)],
                        acc_sh.at[pl.ds(TAIL0, N - TAIL0)])

    plsc.subcore_barrier()

    drain_idx()
    for j in range(LEAD):
        fire_gather(0, j, j)

    def block(b, carry):
        pb = b % IBUF

        @pl.when(b + 2 < nblk)
        def _():
            fire_idx(b + 2, (b + 2) % IBUF)

        for j in range(KSUB):
            drain_gather()
            scale_sub(pb, j, j)
            fire_scatter(pb, j, j)
            if j == 1:
                @pl.when(b < nblk - 1)
                def _():
                    drain_idx()
            if j < RING - LEAD:
                @pl.when(b > 0)
                def _():
                    drain_scatter()
            else:
                drain_scatter()
            jn = j + LEAD
            if jn < KSUB:
                fire_gather(pb, jn, jn)
            else:
                @pl.when(b < nblk - 1)
                def _():
                    fire_gather((b + 1) % IBUF, jn - KSUB, jn - KSUB)
        return carry

    lax.fori_loop(0, nblk, block, 0)
    for _ in range(RING - LEAD):
        drain_scatter()
    plsc.subcore_barrier()

    pltpu.sync_copy(acc_sh.at[pl.ds(row0, ROWS_T)],
                    out_hbm.at[c, pl.ds(row0, ROWS_T)])

    @pl.when(s == NS - 1)
    def _():
        pltpu.sync_copy(acc_sh.at[pl.ds(TAIL0, N - TAIL0)],
                        out_hbm.at[c, pl.ds(TAIL0, N - TAIL0)])

  return _spmm_body


_SC_SCRATCH = [
    pltpu.VMEM((IBUF, KSUB, SUB), jnp.int32),
    pltpu.VMEM((IBUF, KSUB, SUB), jnp.int32),
    pltpu.VMEM((IBUF * CHUNK,), jnp.float32),
    pltpu.VMEM((ZROWS, F), jnp.float32),
    pltpu.VMEM_SHARED((N, F), jnp.float32),
    pltpu.SemaphoreType.DMA,
    pltpu.SemaphoreType.DMA,
    pltpu.SemaphoreType.DMA,
]

_spmm_slab = functools.partial(
    pl.kernel,
    out_type=jax.ShapeDtypeStruct((NC, N, F), jnp.float32),
    mesh=plsc.VectorSubcoreMesh(core_axis_name="c", subcore_axis_name="s",
                                num_cores=NC, num_subcores=NS),
    scratch_types=_SC_SCRATCH,
)(_make_spmm_body(False))

_spmm_dual = functools.partial(
    pl.kernel,
    out_type=jax.ShapeDtypeStruct((NC, N, F), jnp.float32),
    mesh=plsc.VectorSubcoreMesh(core_axis_name="c", subcore_axis_name="s",
                                num_cores=NC, num_subcores=NS),
    scratch_types=_SC_SCRATCH,
)(_make_spmm_body(True))


def _leaky(v):
    return jnp.where(v >= 0, v, NEG_SLOPE * v)


def _dot(a, b):
    return jnp.dot(a, b, preferred_element_type=jnp.float32)


def _stage_a_body(x_ref, wemb_ref, bemb_ref, t1_ref, h_ref, ht1_ref):
    h = _leaky(_dot(x_ref[...], wemb_ref[...]) + bemb_ref[...])
    h_ref[...] = h
    ht1 = _dot(h, t1_ref[...])
    ht1_ref[0] = ht1[:, :F]
    ht1_ref[1] = ht1[:, F:]


def _stage_b1_body(h_ref, pa_ref, t0_ref, b_ref, y_ref, st_ref):
    i = pl.program_id(0)
    s1 = jnp.concatenate([pa_ref[0], pa_ref[1]], axis=1)
    y = _dot(h_ref[...], t0_ref[...]) + s1 + b_ref[...]
    y_ref[...] = y
    part = jnp.concatenate(
        [jnp.sum(y, 0, keepdims=True), jnp.sum(y * y, 0, keepdims=True)], 0)

    @pl.when(i == 0)
    def _():
        st_ref[...] = part

    @pl.when(i > 0)
    def _():
        st_ref[...] = st_ref[...] + part


def _stage_b2_body(y_ref, st_ref, g_ref, be_ref, h_ref, t1_ref,
                   h1_ref, ht1_ref):
    m = st_ref[0:1] * (1.0 / N)
    v = st_ref[1:2] * (1.0 / N) - m * m
    yn = (y_ref[...] - m) / jnp.sqrt(v + EPS) * g_ref[...] + be_ref[...]
    h1 = _leaky(yn)
    h1_ref[...] = h1
    ht1 = _dot(h_ref[...], t1_ref[0:HID]) + _dot(h1, t1_ref[HID:2 * HID])
    ht1_ref[0] = ht1[:, :F]
    ht1_ref[1] = ht1[:, F:]


def _stage_c1_body(h_ref, h1_ref, pa_ref, t0_ref, b_ref,
                   y_ref, st_ref):
    i = pl.program_id(0)
    s2 = jnp.concatenate([pa_ref[0], pa_ref[1]], axis=1)
    y = (_dot(h_ref[...], t0_ref[0:HID]) + _dot(h1_ref[...], t0_ref[HID:])
         + s2 + b_ref[...])
    y_ref[...] = y
    part = jnp.concatenate(
        [jnp.sum(y, 0, keepdims=True), jnp.sum(y * y, 0, keepdims=True)], 0)

    @pl.when(i == 0)
    def _():
        st_ref[...] = part

    @pl.when(i > 0)
    def _():
        st_ref[...] = st_ref[...] + part


def _stage_c2_body(y_ref, st_ref, g_ref, be_ref, h_ref, h1_ref, t1_ref,
                   h2_ref, ht1_ref):
    m = st_ref[0:1] * (1.0 / N)
    v = st_ref[1:2] * (1.0 / N) - m * m
    yn = (y_ref[...] - m) / jnp.sqrt(v + EPS) * g_ref[...] + be_ref[...]
    h2 = _leaky(yn)
    h2_ref[...] = h2
    ht1_ref[...] = (_dot(h_ref[...], t1_ref[0:HID])
                    + _dot(h1_ref[...], t1_ref[HID:2 * HID])
                    + _dot(h2, t1_ref[2 * HID:]))


def _stage_d_body(h_ref, h1_ref, h2_ref, p_ref, t0_ref, out_ref):
    out_ref[...] = (_dot(h_ref[...], t0_ref[0:HID])
                    + _dot(h1_ref[...], t0_ref[HID:2 * HID])
                    + _dot(h2_ref[...], t0_ref[2 * HID:])
                    + p_ref[0] + p_ref[1])


def _row_spec(width):
    return pl.BlockSpec((BLK, width), lambda i: (i, 0))


def _part_spec():
    return pl.BlockSpec((NC, BLK, F), lambda i: (0, i, 0))


def _full_spec(shape):
    return pl.BlockSpec(shape, lambda i: tuple(0 for _ in shape))


def _stats_spec():
    return pl.BlockSpec((2, HID), lambda i: (0, 0))


def kernel(x, edge_index, edge_weight, g_size, W_emb, b_emb, gc1_T0, gc1_T1,
           gc1_b, gc1_gamma, gc1_beta, gc2_T0, gc2_T1, gc2_b, gc2_gamma,
           gc2_beta, gcl_T0, gcl_T1):
    src2d = edge_index[0].reshape(E // SUB, SUB)
    dst2d = edge_index[1].reshape(E // SUB, SUB)
    f32 = jnp.float32

    h, h1t1 = pl.pallas_call(
        _stage_a_body,
        grid=(GRID,),
        in_specs=[_row_spec(IN_FEAT), _full_spec((IN_FEAT, HID)),
                  _full_spec((1, HID)), _full_spec((HID, HID))],
        out_specs=[_row_spec(HID), _part_spec()],
        out_shape=[jax.ShapeDtypeStruct((N, HID), f32),
                   jax.ShapeDtypeStruct((NC, N, F), f32)],
    )(x, W_emb, b_emb.reshape(1, HID), gc1_T1)

    s1 = _spmm_dual(h1t1, src2d, dst2d, edge_weight)

    y1, st1 = pl.pallas_call(
        _stage_b1_body,
        grid=(GRID,),
        in_specs=[_row_spec(HID), _part_spec(),
                  _full_spec((HID, HID)), _full_spec((1, HID))],
        out_specs=[_row_spec(HID), _stats_spec()],
        out_shape=[jax.ShapeDtypeStruct((N, HID), f32),
                   jax.ShapeDtypeStruct((2, HID), f32)],
    )(h, s1, gc1_T0, gc1_b.reshape(1, HID))

    h1, h2t1 = pl.pallas_call(
        _stage_b2_body,
        grid=(GRID,),
        in_specs=[_row_spec(HID), _stats_spec(), _full_spec((1, HID)),
                  _full_spec((1, HID)), _row_spec(HID),
                  _full_spec((2 * HID, HID))],
        out_specs=[_row_spec(HID), _part_spec()],
        out_shape=[jax.ShapeDtypeStruct((N, HID), f32),
                   jax.ShapeDtypeStruct((NC, N, F), f32)],
    )(y1, st1, gc1_gamma.reshape(1, HID), gc1_beta.reshape(1, HID), h,
      gc2_T1)

    s2 = _spmm_dual(h2t1, src2d, dst2d, edge_weight)

    y2, st2 = pl.pallas_call(
        _stage_c1_body,
        grid=(GRID,),
        in_specs=[_row_spec(HID), _row_spec(HID), _part_spec(),
                  _full_spec((2 * HID, HID)), _full_spec((1, HID))],
        out_specs=[_row_spec(HID), _stats_spec()],
        out_shape=[jax.ShapeDtypeStruct((N, HID), f32),
                   jax.ShapeDtypeStruct((2, HID), f32)],
    )(h, h1, s2, gc2_T0, gc2_b.reshape(1, HID))

    h2, h3t1 = pl.pallas_call(
        _stage_c2_body,
        grid=(GRID,),
        in_specs=[_row_spec(HID), _stats_spec(), _full_spec((1, HID)),
                  _full_spec((1, HID)), _row_spec(HID), _row_spec(HID),
                  _full_spec((3 * HID, OUT_FEAT))],
        out_specs=[_row_spec(HID), _row_spec(OUT_FEAT)],
        out_shape=[jax.ShapeDtypeStruct((N, HID), f32),
                   jax.ShapeDtypeStruct((N, OUT_FEAT), f32)],
    )(y2, st2, gc2_gamma.reshape(1, HID), gc2_beta.reshape(1, HID), h, h1,
      gcl_T1)

    p3 = _spmm_slab(h3t1, src2d, dst2d, edge_weight)

    out = pl.pallas_call(
        _stage_d_body,
        grid=(GRID,),
        in_specs=[_row_spec(HID), _row_spec(HID), _row_spec(HID),
                  _part_spec(), _full_spec((3 * HID, OUT_FEAT))],
        out_specs=_row_spec(OUT_FEAT),
        out_shape=jax.ShapeDtypeStruct((N, OUT_FEAT), f32),
    )(h, h1, h2, p3, gcl_T0)

    return out


# speedup vs baseline: 12.9352x; 1.0089x over previous
import functools

import jax
import jax.numpy as jnp
from jax import lax
from jax.experimental import pallas as pl
from jax.experimental.pallas import tpu as pltpu
from jax.experimental.pallas import tpu_sc as plsc

N = 10000
E = 320000
IN_FEAT = 128
OUT_FEAT = 128
HID = 256
NEG_SLOPE = 0.01
EPS = 1e-5

F = 128
L = 16
NC = 2
NS = 16
NW = NC * NS
EB = E // NW
SUB = 25
KSUB = 8
CHUNK = SUB * KSUB
NBLK = EB // CHUNK
SB = 5
NSB = NBLK // SB
H = CHUNK // 2
KH = KSUB // 2
ROWS_T = 624
TAIL0 = NS * ROWS_T

BLK = 2000
GRID = N // BLK


NT = NBLK * KSUB
RING = KSUB
SLOTR = 32
ZROWS = RING * SLOTR
LEAD = 6
IBUF = 4


def _make_spmm_body(dual):
  nblk = E // ((NS if dual else NW) * CHUNK)

  def _spmm_body(xw_hbm, src_hbm, dst_hbm, w_hbm, out_hbm,
                 src_v, dst_v, w_v, rows_v, acc_sh, sem_g, sem_s, sem_i):
    c = lax.axis_index("c")
    s = lax.axis_index("s")
    wid = s if dual else c * NS + s
    xw = xw_hbm.at[c] if dual else xw_hbm

    def fire_idx(b, pset):
        r0 = pl.multiple_of((wid * nblk + b) * KSUB, 8)
        e0 = pl.multiple_of((wid * nblk + b) * CHUNK, 8)
        pltpu.async_copy(src_hbm.at[pl.ds(r0, KSUB)], src_v.at[pset], sem_i)
        pltpu.async_copy(dst_hbm.at[pl.ds(r0, KSUB)], dst_v.at[pset], sem_i)
        pltpu.async_copy(
            w_hbm.at[pl.ds(e0, CHUNK)],
            w_v.at[pl.ds(pl.multiple_of(pset * CHUNK, 8), CHUNK)], sem_i)

    def drain_idx():
        pltpu.make_async_copy(src_hbm.at[pl.ds(0, KSUB)], src_v.at[0],
                              sem_i).wait()
        pltpu.make_async_copy(dst_hbm.at[pl.ds(0, KSUB)], dst_v.at[0],
                              sem_i).wait()
        pltpu.make_async_copy(w_hbm.at[pl.ds(0, CHUNK)],
                              w_v.at[pl.ds(0, CHUNK)], sem_i).wait()

    def fire_gather(pset, j, slot):
        pltpu.async_copy(xw.at[src_v.at[pset, j]],
                         rows_v.at[pl.ds(slot * SLOTR, SUB)], sem_g)

    def drain_gather():
        pltpu.make_async_copy(xw.at[src_v.at[0, 0]],
                              rows_v.at[pl.ds(0, SUB)], sem_g).wait()

    def fire_scatter(pset, j, slot):
        pltpu.async_copy(rows_v.at[pl.ds(slot * SLOTR, SUB)],
                         acc_sh.at[dst_v.at[pset, j]], sem_s, add=True)

    def drain_scatter():
        pltpu.make_async_copy(rows_v.at[pl.ds(0, SUB)],
                              acc_sh.at[dst_v.at[0, 0]], sem_s).wait()

    def scale_sub(pset, j, slot):
        base_w = pset * CHUNK + j * SUB
        base_e = slot * SLOTR
        w16 = w_v[pl.ds(base_w, L)]
        for i in range(L):
            w_s = w16[i]
            e = base_e + i
            for k in range(F // L):
                sl = pl.ds(k * L, L)
                rows_v[e, sl] = rows_v[e, sl] * w_s
        w16 = w_v[pl.ds(base_w + SUB - L, L)]
        for i in range(2 * L - SUB, L):
            w_s = w16[i]
            e = base_e + (SUB - L) + i
            for k in range(F // L):
                sl = pl.ds(k * L, L)
                rows_v[e, sl] = rows_v[e, sl] * w_s

    fire_idx(0, 0)
    fire_idx(1, 1)
    zeros = jnp.zeros((L,), jnp.float32)

    def zero_row(e, carry):
        for k in range(F // L):
            rows_v[e, pl.ds(k * L, L)] = zeros
        return carry

    lax.fori_loop(0, ZROWS, zero_row, 0)
    row0 = pl.multiple_of(s * ROWS_T, 8)
    for t in range(ROWS_T // ZROWS):
        pltpu.sync_copy(rows_v, acc_sh.at[pl.ds(row0 + t * ZROWS, ZROWS)])
    _rem = ROWS_T % ZROWS
    if _rem:
        pltpu.sync_copy(
            rows_v.at[pl.ds(0, _rem)],
            acc_sh.at[pl.ds(row0 + (ROWS_T // ZROWS) * ZROWS, _rem)])

    @pl.when(s == NS - 1)
    def _():
        pltpu.sync_copy(rows_v.at[pl.ds(0, N - TAIL0)],
                        acc_sh.at[pl.ds(TAIL0, N - TAIL0)])

    plsc.subcore_barrier()

    drain_idx()
    for j in range(LEAD):
        fire_gather(0, j, j)

    def block(b, carry):
        pb = b % IBUF

        @pl.when(b + 2 < nblk)
        def _():
            fire_idx(b + 2, (b + 2) % IBUF)

        for j in range(KSUB):
            drain_gather()
            scale_sub(pb, j, j)
            fire_scatter(pb, j, j)
            if j == 1:
                @pl.when(b < nblk - 1)
                def _():
                    drain_idx()
            if j < RING - LEAD:
                @pl.when(b > 0)
                def _():
                    drain_scatter()
            else:
                drain_scatter()
            jn = j + LEAD
            if jn < KSUB:
                fire_gather(pb, jn, jn)
            else:
                @pl.when(b < nblk - 1)
                def _():
                    fire_gather((b + 1) % IBUF, jn - KSUB, jn - KSUB)
        return carry

    lax.fori_loop(0, nblk, block, 0)
    for _ in range(RING - LEAD):
        drain_scatter()
    plsc.subcore_barrier()

    pltpu.sync_copy(acc_sh.at[pl.ds(row0, ROWS_T)],
                    out_hbm.at[c, pl.ds(row0, ROWS_T)])

    @pl.when(s == NS - 1)
    def _():
        pltpu.sync_copy(acc_sh.at[pl.ds(TAIL0, N - TAIL0)],
                        out_hbm.at[c, pl.ds(TAIL0, N - TAIL0)])

  return _spmm_body


_SC_SCRATCH = [
    pltpu.VMEM((IBUF, KSUB, SUB), jnp.int32),
    pltpu.VMEM((IBUF, KSUB, SUB), jnp.int32),
    pltpu.VMEM((IBUF * CHUNK,), jnp.float32),
    pltpu.VMEM((ZROWS, F), jnp.float32),
    pltpu.VMEM_SHARED((N, F), jnp.float32),
    pltpu.SemaphoreType.DMA,
    pltpu.SemaphoreType.DMA,
    pltpu.SemaphoreType.DMA,
]

_spmm_slab = functools.partial(
    pl.kernel,
    out_type=jax.ShapeDtypeStruct((NC, N, F), jnp.float32),
    mesh=plsc.VectorSubcoreMesh(core_axis_name="c", subcore_axis_name="s",
                                num_cores=NC, num_subcores=NS),
    scratch_types=_SC_SCRATCH,
)(_make_spmm_body(False))

_spmm_dual = functools.partial(
    pl.kernel,
    out_type=jax.ShapeDtypeStruct((NC, N, F), jnp.float32),
    mesh=plsc.VectorSubcoreMesh(core_axis_name="c", subcore_axis_name="s",
                                num_cores=NC, num_subcores=NS),
    scratch_types=_SC_SCRATCH,
)(_make_spmm_body(True))


def _leaky(v):
    return jnp.where(v >= 0, v, NEG_SLOPE * v)


def _dot(a, b):
    return jnp.dot(a, b, preferred_element_type=jnp.float32)


def _stage_a_body(x_ref, wemb_ref, bemb_ref, t1_ref, h_ref, ht1_ref):
    h = _leaky(_dot(x_ref[...], wemb_ref[...]) + bemb_ref[...])
    h_ref[...] = h
    ht1 = _dot(h, t1_ref[...])
    ht1_ref[0] = ht1[:, :F]
    ht1_ref[1] = ht1[:, F:]


def _stage_b1pre_body(h_ref, t0_ref, b_ref, z_ref):
    z_ref[...] = _dot(h_ref[...], t0_ref[...]) + b_ref[...]


def _stage_c1pre_body(h_ref, h1_ref, t0_ref, b_ref, z_ref):
    z_ref[...] = (_dot(h_ref[...], t0_ref[0:HID])
                  + _dot(h1_ref[...], t0_ref[HID:]) + b_ref[...])


def _stage_dpre_body(h_ref, h1_ref, h2_ref, t0_ref, z_ref):
    z_ref[...] = (_dot(h_ref[...], t0_ref[0:HID])
                  + _dot(h1_ref[...], t0_ref[HID:2 * HID])
                  + _dot(h2_ref[...], t0_ref[2 * HID:]))


def _stage_post_body(z_ref, pa_ref, y_ref, st_ref):
    i = pl.program_id(0)
    y = z_ref[...] + jnp.concatenate([pa_ref[0], pa_ref[1]], axis=1)
    y_ref[...] = y
    part = jnp.concatenate(
        [jnp.sum(y, 0, keepdims=True), jnp.sum(y * y, 0, keepdims=True)], 0)

    @pl.when(i == 0)
    def _():
        st_ref[...] = part

    @pl.when(i > 0)
    def _():
        st_ref[...] = st_ref[...] + part


def _stage_b2_body(y_ref, st_ref, g_ref, be_ref, h_ref, t1_ref,
                   h1_ref, ht1_ref):
    m = st_ref[0:1] * (1.0 / N)
    v = st_ref[1:2] * (1.0 / N) - m * m
    yn = (y_ref[...] - m) / jnp.sqrt(v + EPS) * g_ref[...] + be_ref[...]
    h1 = _leaky(yn)
    h1_ref[...] = h1
    ht1 = _dot(h_ref[...], t1_ref[0:HID]) + _dot(h1, t1_ref[HID:2 * HID])
    ht1_ref[0] = ht1[:, :F]
    ht1_ref[1] = ht1[:, F:]


def _stage_c2_body(y_ref, st_ref, g_ref, be_ref, h_ref, h1_ref, t1_ref,
                   h2_ref, ht1_ref):
    m = st_ref[0:1] * (1.0 / N)
    v = st_ref[1:2] * (1.0 / N) - m * m
    yn = (y_ref[...] - m) / jnp.sqrt(v + EPS) * g_ref[...] + be_ref[...]
    h2 = _leaky(yn)
    h2_ref[...] = h2
    ht1_ref[...] = (_dot(h_ref[...], t1_ref[0:HID])
                    + _dot(h1_ref[...], t1_ref[HID:2 * HID])
                    + _dot(h2, t1_ref[2 * HID:]))


def _stage_dpost_body(z_ref, p_ref, out_ref):
    out_ref[...] = z_ref[...] + p_ref[0] + p_ref[1]


def _row_spec(width):
    return pl.BlockSpec((BLK, width), lambda i: (i, 0))


def _part_spec():
    return pl.BlockSpec((NC, BLK, F), lambda i: (0, i, 0))


def _full_spec(shape):
    return pl.BlockSpec(shape, lambda i: tuple(0 for _ in shape))


def _stats_spec():
    return pl.BlockSpec((2, HID), lambda i: (0, 0))


def kernel(x, edge_index, edge_weight, g_size, W_emb, b_emb, gc1_T0, gc1_T1,
           gc1_b, gc1_gamma, gc1_beta, gc2_T0, gc2_T1, gc2_b, gc2_gamma,
           gc2_beta, gcl_T0, gcl_T1):
    src2d = edge_index[0].reshape(E // SUB, SUB)
    dst2d = edge_index[1].reshape(E // SUB, SUB)
    f32 = jnp.float32

    h, h1t1 = pl.pallas_call(
        _stage_a_body,
        grid=(GRID,),
        in_specs=[_row_spec(IN_FEAT), _full_spec((IN_FEAT, HID)),
                  _full_spec((1, HID)), _full_spec((HID, HID))],
        out_specs=[_row_spec(HID), _part_spec()],
        out_shape=[jax.ShapeDtypeStruct((N, HID), f32),
                   jax.ShapeDtypeStruct((NC, N, F), f32)],
    )(x, W_emb, b_emb.reshape(1, HID), gc1_T1)

    z1 = pl.pallas_call(
        _stage_b1pre_body,
        grid=(GRID,),
        in_specs=[_row_spec(HID), _full_spec((HID, HID)),
                  _full_spec((1, HID))],
        out_specs=_row_spec(HID),
        out_shape=jax.ShapeDtypeStruct((N, HID), f32),
    )(h, gc1_T0, gc1_b.reshape(1, HID))

    s1 = _spmm_dual(h1t1, src2d, dst2d, edge_weight)

    y1, st1 = pl.pallas_call(
        _stage_post_body,
        grid=(GRID,),
        in_specs=[_row_spec(HID), _part_spec()],
        out_specs=[_row_spec(HID), _stats_spec()],
        out_shape=[jax.ShapeDtypeStruct((N, HID), f32),
                   jax.ShapeDtypeStruct((2, HID), f32)],
    )(z1, s1)

    h1, h2t1 = pl.pallas_call(
        _stage_b2_body,
        grid=(GRID,),
        in_specs=[_row_spec(HID), _stats_spec(), _full_spec((1, HID)),
                  _full_spec((1, HID)), _row_spec(HID),
                  _full_spec((2 * HID, HID))],
        out_specs=[_row_spec(HID), _part_spec()],
        out_shape=[jax.ShapeDtypeStruct((N, HID), f32),
                   jax.ShapeDtypeStruct((NC, N, F), f32)],
    )(y1, st1, gc1_gamma.reshape(1, HID), gc1_beta.reshape(1, HID), h,
      gc2_T1)

    z2 = pl.pallas_call(
        _stage_c1pre_body,
        grid=(GRID,),
        in_specs=[_row_spec(HID), _row_spec(HID),
                  _full_spec((2 * HID, HID)), _full_spec((1, HID))],
        out_specs=_row_spec(HID),
        out_shape=jax.ShapeDtypeStruct((N, HID), f32),
    )(h, h1, gc2_T0, gc2_b.reshape(1, HID))

    s2 = _spmm_dual(h2t1, src2d, dst2d, edge_weight)

    y2, st2 = pl.pallas_call(
        _stage_post_body,
        grid=(GRID,),
        in_specs=[_row_spec(HID), _part_spec()],
        out_specs=[_row_spec(HID), _stats_spec()],
        out_shape=[jax.ShapeDtypeStruct((N, HID), f32),
                   jax.ShapeDtypeStruct((2, HID), f32)],
    )(z2, s2)

    h2, h3t1 = pl.pallas_call(
        _stage_c2_body,
        grid=(GRID,),
        in_specs=[_row_spec(HID), _stats_spec(), _full_spec((1, HID)),
                  _full_spec((1, HID)), _row_spec(HID), _row_spec(HID),
                  _full_spec((3 * HID, OUT_FEAT))],
        out_specs=[_row_spec(HID), _row_spec(OUT_FEAT)],
        out_shape=[jax.ShapeDtypeStruct((N, HID), f32),
                   jax.ShapeDtypeStruct((N, OUT_FEAT), f32)],
    )(y2, st2, gc2_gamma.reshape(1, HID), gc2_beta.reshape(1, HID), h, h1,
      gcl_T1)

    zd = pl.pallas_call(
        _stage_dpre_body,
        grid=(GRID,),
        in_specs=[_row_spec(HID), _row_spec(HID), _row_spec(HID),
                  _full_spec((3 * HID, OUT_FEAT))],
        out_specs=_row_spec(OUT_FEAT),
        out_shape=jax.ShapeDtypeStruct((N, OUT_FEAT), f32),
    )(h, h1, h2, gcl_T0)

    p3 = _spmm_slab(h3t1, src2d, dst2d, edge_weight)

    out = pl.pallas_call(
        _stage_dpost_body,
        grid=(GRID,),
        in_specs=[_row_spec(OUT_FEAT), _part_spec()],
        out_specs=_row_spec(OUT_FEAT),
        out_shape=jax.ShapeDtypeStruct((N, OUT_FEAT), f32),
    )(zd, p3)

    return out
